# baseline (speedup 1.0000x reference)
import functools

import jax
import jax.numpy as jnp
from jax import lax
from jax.experimental import pallas as pl
from jax.experimental.pallas import tpu as pltpu
from jax.experimental.pallas import tpu_sc as plsc

VOCAB = 100000
EMBED = 300
HIDDEN = 10
NUM_CLASS = 10
SEQ = 50
BATCH = 4096

DPAD = 16
VBLK = 10000

NC = 2
NS = 16
NW = NC * NS
BPW = BATCH // NW


def _fold_body(emb_ref, w1_ref, w2p_ref, out_ref):
    e = emb_ref[...]
    h = lax.dot_general(e, w1_ref[...], (((1,), (1,)), ((), ())),
                        preferred_element_type=jnp.float32)
    p = lax.dot_general(h, w2p_ref[...], (((1,), (1,)), ((), ())),
                        preferred_element_type=jnp.float32)
    out_ref[...] = p * (1.0 / SEQ)


def _fold_table(emb, w1, w2p):
    return pl.pallas_call(
        _fold_body,
        grid=(VOCAB // VBLK,),
        in_specs=[
            pl.BlockSpec((VBLK, EMBED), lambda i: (i, 0)),
            pl.BlockSpec((HIDDEN, EMBED), lambda i: (0, 0)),
            pl.BlockSpec((DPAD, HIDDEN), lambda i: (0, 0)),
        ],
        out_specs=pl.BlockSpec((VBLK, DPAD), lambda i: (i, 0)),
        out_shape=jax.ShapeDtypeStruct((VOCAB, DPAD), jnp.float32),
    )(emb, w1, w2p)


_sc_mesh = plsc.VectorSubcoreMesh(core_axis_name="c", subcore_axis_name="s")


SCH = 10
NCHUNK = SEQ // SCH


@functools.partial(
    pl.kernel,
    out_type=jax.ShapeDtypeStruct((BATCH, DPAD), jnp.float32),
    mesh=_sc_mesh,
    compiler_params=pltpu.CompilerParams(use_tc_tiling_on_sc=False),
    scratch_types=[
        pltpu.VMEM((SEQ, BPW), jnp.int32),
        pltpu.VMEM((2, SCH, BPW, DPAD), jnp.float32),
        pltpu.VMEM((BPW, DPAD), jnp.float32),
        pltpu.VMEM((DPAD,), jnp.float32),
        pltpu.SemaphoreType.DMA,
        pltpu.SemaphoreType.DMA,
    ],
)
def _pool_kernel(tab_hbm, x_hbm, bias_hbm, out_hbm,
                 idx_v, rows_v, res_v, bias_v, sem0, sem1):
    wid = lax.axis_index("s") * NC + lax.axis_index("c")
    base = wid * BPW
    pltpu.sync_copy(x_hbm.at[:, pl.ds(base, BPW)], idx_v)
    pltpu.sync_copy(bias_hbm, bias_v)
    sems = (sem0, sem1)

    def fire(chunk):
        buf = chunk % 2
        for j in range(SCH):
            s = chunk * SCH + j
            pltpu.async_copy(tab_hbm.at[idx_v.at[s]], rows_v.at[buf, j],
                             sems[buf])

    def drain(chunk):
        buf = chunk % 2
        for j in range(SCH):
            s = chunk * SCH + j
            pltpu.make_async_copy(tab_hbm.at[idx_v.at[s]], rows_v.at[buf, j],
                                  sems[buf]).wait()

    bias = bias_v[...]

    def reduce_chunk(chunk, init):
        buf = chunk % 2

        def reduce_b(b, c):
            p0 = rows_v[buf, 0, b]
            p1 = rows_v[buf, 1, b]
            for j in range(2, SCH, 2):
                p0 = p0 + rows_v[buf, j, b]
                p1 = p1 + rows_v[buf, j + 1, b]
            if init:
                res_v[b] = (p0 + p1) + bias
            else:
                res_v[b] = (p0 + p1) + res_v[b]
            return c

        lax.fori_loop(0, BPW, reduce_b, 0)

    fire(0)
    for chunk in range(NCHUNK):
        if chunk + 1 < NCHUNK:
            fire(chunk + 1)
        drain(chunk)
        reduce_chunk(chunk, init=(chunk == 0))
    pltpu.sync_copy(res_v, out_hbm.at[pl.ds(base, BPW)])


def kernel(x, emb, W1, b1, W2, b2):
    x = x.astype(jnp.int32)
    w2p = jnp.pad(W2, ((0, DPAD - NUM_CLASS), (0, 0)))
    bias = jnp.pad(W2 @ b1 + b2, (0, DPAD - NUM_CLASS))
    tab = _fold_table(emb, W1, w2p)
    return _pool_kernel(tab, x, bias)[:, :NUM_CLASS]


# speedup vs baseline: 1.7444x; 1.7444x over previous
import functools

import jax
import jax.numpy as jnp
from jax import lax
from jax.experimental import pallas as pl
from jax.experimental.pallas import tpu as pltpu
from jax.experimental.pallas import tpu_sc as plsc

VOCAB = 100000
EMBED = 300
HIDDEN = 10
NUM_CLASS = 10
SEQ = 50
BATCH = 4096

DPAD = 16
LBLK = 2048

NC = 2
NS = 16
NW = NC * NS
BPW = BATCH // NW


def _fold_body(embT_ref, w1_ref, w2p_ref, out_ref):
    h = lax.dot_general(embT_ref[...], w1_ref[...], (((0,), (1,)), ((), ())),
                        preferred_element_type=jnp.float32)
    p = lax.dot_general(h, w2p_ref[...], (((1,), (1,)), ((), ())),
                        preferred_element_type=jnp.float32)
    out_ref[...] = p * (1.0 / SEQ)


def _fold_table(embT, w1, w2p):
    grid = (VOCAB + LBLK - 1) // LBLK
    return pl.pallas_call(
        _fold_body,
        grid=(grid,),
        in_specs=[
            pl.BlockSpec((EMBED, LBLK), lambda i: (0, i)),
            pl.BlockSpec((HIDDEN, EMBED), lambda i: (0, 0)),
            pl.BlockSpec((DPAD, HIDDEN), lambda i: (0, 0)),
        ],
        out_specs=pl.BlockSpec((LBLK, DPAD), lambda i: (i, 0)),
        out_shape=jax.ShapeDtypeStruct((VOCAB, DPAD), jnp.float32),
    )(embT, w1, w2p)


_sc_mesh = plsc.VectorSubcoreMesh(core_axis_name="c", subcore_axis_name="s")


SCH = 10
NCHUNK = SEQ // SCH


@functools.partial(
    pl.kernel,
    out_type=jax.ShapeDtypeStruct((BATCH, DPAD), jnp.float32),
    mesh=_sc_mesh,
    compiler_params=pltpu.CompilerParams(use_tc_tiling_on_sc=False),
    scratch_types=[
        pltpu.VMEM((SEQ, BPW), jnp.int32),
        pltpu.VMEM((2, SCH, BPW, DPAD), jnp.float32),
        pltpu.VMEM((BPW, DPAD), jnp.float32),
        pltpu.VMEM((DPAD,), jnp.float32),
        pltpu.SemaphoreType.DMA,
        pltpu.SemaphoreType.DMA,
    ],
)
def _pool_kernel(tab_hbm, x_hbm, bias_hbm, out_hbm,
                 idx_v, rows_v, res_v, bias_v, sem0, sem1):
    wid = lax.axis_index("s") * NC + lax.axis_index("c")
    base = wid * BPW
    pltpu.sync_copy(x_hbm.at[:, pl.ds(base, BPW)], idx_v)
    pltpu.sync_copy(bias_hbm, bias_v)
    sems = (sem0, sem1)

    def fire(chunk):
        buf = chunk % 2
        for j in range(SCH):
            s = chunk * SCH + j
            pltpu.async_copy(tab_hbm.at[idx_v.at[s]], rows_v.at[buf, j],
                             sems[buf])

    def drain(chunk):
        buf = chunk % 2
        for j in range(SCH):
            s = chunk * SCH + j
            pltpu.make_async_copy(tab_hbm.at[idx_v.at[s]], rows_v.at[buf, j],
                                  sems[buf]).wait()

    bias = bias_v[...]

    def reduce_chunk(chunk, init):
        buf = chunk % 2

        def reduce_b(b, c):
            p0 = rows_v[buf, 0, b]
            p1 = rows_v[buf, 1, b]
            for j in range(2, SCH, 2):
                p0 = p0 + rows_v[buf, j, b]
                p1 = p1 + rows_v[buf, j + 1, b]
            if init:
                res_v[b] = (p0 + p1) + bias
            else:
                res_v[b] = (p0 + p1) + res_v[b]
            return c

        lax.fori_loop(0, BPW, reduce_b, 0)

    fire(0)
    for chunk in range(NCHUNK):
        if chunk + 1 < NCHUNK:
            fire(chunk + 1)
        drain(chunk)
        reduce_chunk(chunk, init=(chunk == 0))
    pltpu.sync_copy(res_v, out_hbm.at[pl.ds(base, BPW)])


def kernel(x, emb, W1, b1, W2, b2):
    x = x.astype(jnp.int32)
    w2p = jnp.pad(W2, ((0, DPAD - NUM_CLASS), (0, 0)))
    bias = jnp.pad(W2 @ b1 + b2, (0, DPAD - NUM_CLASS))
    tab = _fold_table(emb.T, W1, w2p)
    return _pool_kernel(tab, x, bias)[:, :NUM_CLASS]


# speedup vs baseline: 1.9602x; 1.1237x over previous
import functools

import jax
import jax.numpy as jnp
from jax import lax
from jax.experimental import pallas as pl
from jax.experimental.pallas import tpu as pltpu
from jax.experimental.pallas import tpu_sc as plsc

VOCAB = 100000
EMBED = 300
HIDDEN = 10
NUM_CLASS = 10
SEQ = 50
BATCH = 4096

DPAD = 16
LBLK = 4096

NC = 2
NS = 16
NW = NC * NS
BPW = BATCH // NW


def _fold_body(embT_ref, w1_ref, w2p_ref, out_ref):
    h = lax.dot_general(embT_ref[...], w1_ref[...], (((0,), (1,)), ((), ())),
                        preferred_element_type=jnp.float32)
    p = lax.dot_general(h, w2p_ref[...], (((1,), (1,)), ((), ())),
                        preferred_element_type=jnp.float32)
    out_ref[...] = p * (1.0 / SEQ)


def _fold_table(embT, w1, w2p):
    grid = (VOCAB + LBLK - 1) // LBLK
    return pl.pallas_call(
        _fold_body,
        grid=(grid,),
        in_specs=[
            pl.BlockSpec((EMBED, LBLK), lambda i: (0, i)),
            pl.BlockSpec((HIDDEN, EMBED), lambda i: (0, 0)),
            pl.BlockSpec((DPAD, HIDDEN), lambda i: (0, 0)),
        ],
        out_specs=pl.BlockSpec((LBLK, DPAD), lambda i: (i, 0)),
        out_shape=jax.ShapeDtypeStruct((VOCAB, DPAD), jnp.float32),
    )(embT, w1, w2p)


_sc_mesh = plsc.VectorSubcoreMesh(core_axis_name="c", subcore_axis_name="s")


SCH = 10
NCHUNK = SEQ // SCH


@functools.partial(
    pl.kernel,
    out_type=jax.ShapeDtypeStruct((BATCH, DPAD), jnp.float32),
    mesh=_sc_mesh,
    compiler_params=pltpu.CompilerParams(use_tc_tiling_on_sc=False),
    scratch_types=[
        pltpu.VMEM((SEQ, BPW), jnp.int32),
        pltpu.VMEM((2, SCH, BPW, DPAD), jnp.float32),
        pltpu.VMEM((BPW, DPAD), jnp.float32),
        pltpu.VMEM((DPAD,), jnp.float32),
        pltpu.SemaphoreType.DMA,
        pltpu.SemaphoreType.DMA,
    ],
)
def _pool_kernel(tab_hbm, x_hbm, bias_hbm, out_hbm,
                 idx_v, rows_v, res_v, bias_v, sem0, sem1):
    wid = lax.axis_index("s") * NC + lax.axis_index("c")
    base = wid * BPW
    pltpu.sync_copy(x_hbm.at[:, pl.ds(base, BPW)], idx_v)
    pltpu.sync_copy(bias_hbm, bias_v)
    sems = (sem0, sem1)

    def fire(chunk):
        buf = chunk % 2
        for j in range(SCH):
            s = chunk * SCH + j
            pltpu.async_copy(tab_hbm.at[idx_v.at[s]], rows_v.at[buf, j],
                             sems[buf])

    def drain(chunk):
        buf = chunk % 2
        for j in range(SCH):
            s = chunk * SCH + j
            pltpu.make_async_copy(tab_hbm.at[idx_v.at[s]], rows_v.at[buf, j],
                                  sems[buf]).wait()

    bias = bias_v[...]

    def reduce_chunk(chunk, init):
        buf = chunk % 2

        def reduce_b(b, c):
            p0 = rows_v[buf, 0, b]
            p1 = rows_v[buf, 1, b]
            for j in range(2, SCH, 2):
                p0 = p0 + rows_v[buf, j, b]
                p1 = p1 + rows_v[buf, j + 1, b]
            if init:
                res_v[b] = (p0 + p1) + bias
            else:
                res_v[b] = (p0 + p1) + res_v[b]
            return c

        lax.fori_loop(0, BPW, reduce_b, 0)

    fire(0)
    for chunk in range(NCHUNK):
        if chunk + 1 < NCHUNK:
            fire(chunk + 1)
        drain(chunk)
        reduce_chunk(chunk, init=(chunk == 0))
    pltpu.sync_copy(res_v, out_hbm.at[pl.ds(base, BPW)])


def kernel(x, emb, W1, b1, W2, b2):
    x = x.astype(jnp.int32)
    w2p = jnp.pad(W2, ((0, DPAD - NUM_CLASS), (0, 0)))
    bias = jnp.pad(W2 @ b1 + b2, (0, DPAD - NUM_CLASS))
    tab = _fold_table(emb.T, W1, w2p)
    return _pool_kernel(tab, x, bias)[:, :NUM_CLASS]


# speedup vs baseline: 2.0010x; 1.0208x over previous
import functools

import jax
import jax.numpy as jnp
from jax import lax
from jax.experimental import pallas as pl
from jax.experimental.pallas import tpu as pltpu
from jax.experimental.pallas import tpu_sc as plsc

VOCAB = 100000
EMBED = 300
HIDDEN = 10
NUM_CLASS = 10
SEQ = 50
BATCH = 4096

DPAD = 16
LBLK = 8192

NC = 2
NS = 16
NW = NC * NS
BPW = BATCH // NW


def _fold_body(embT_ref, w1_ref, w2p_ref, out_ref):
    h = lax.dot_general(embT_ref[...], w1_ref[...], (((0,), (1,)), ((), ())),
                        preferred_element_type=jnp.float32)
    p = lax.dot_general(h, w2p_ref[...], (((1,), (1,)), ((), ())),
                        preferred_element_type=jnp.float32)
    out_ref[...] = p * (1.0 / SEQ)


def _fold_table(embT, w1, w2p):
    grid = (VOCAB + LBLK - 1) // LBLK
    return pl.pallas_call(
        _fold_body,
        grid=(grid,),
        in_specs=[
            pl.BlockSpec((EMBED, LBLK), lambda i: (0, i)),
            pl.BlockSpec((HIDDEN, EMBED), lambda i: (0, 0)),
            pl.BlockSpec((DPAD, HIDDEN), lambda i: (0, 0)),
        ],
        out_specs=pl.BlockSpec((LBLK, DPAD), lambda i: (i, 0)),
        out_shape=jax.ShapeDtypeStruct((VOCAB, DPAD), jnp.float32),
    )(embT, w1, w2p)


_sc_mesh = plsc.VectorSubcoreMesh(core_axis_name="c", subcore_axis_name="s")


SCH = 10
NCHUNK = SEQ // SCH


@functools.partial(
    pl.kernel,
    out_type=jax.ShapeDtypeStruct((BATCH, DPAD), jnp.float32),
    mesh=_sc_mesh,
    compiler_params=pltpu.CompilerParams(use_tc_tiling_on_sc=False),
    scratch_types=[
        pltpu.VMEM((SEQ, BPW), jnp.int32),
        pltpu.VMEM((2, SCH, BPW, DPAD), jnp.float32),
        pltpu.VMEM((BPW, DPAD), jnp.float32),
        pltpu.VMEM((DPAD,), jnp.float32),
        pltpu.SemaphoreType.DMA,
        pltpu.SemaphoreType.DMA,
    ],
)
def _pool_kernel(tab_hbm, x_hbm, bias_hbm, out_hbm,
                 idx_v, rows_v, res_v, bias_v, sem0, sem1):
    wid = lax.axis_index("s") * NC + lax.axis_index("c")
    base = wid * BPW
    pltpu.sync_copy(x_hbm.at[:, pl.ds(base, BPW)], idx_v)
    pltpu.sync_copy(bias_hbm, bias_v)
    sems = (sem0, sem1)

    def fire(chunk):
        buf = chunk % 2
        for j in range(SCH):
            s = chunk * SCH + j
            pltpu.async_copy(tab_hbm.at[idx_v.at[s]], rows_v.at[buf, j],
                             sems[buf])

    def drain(chunk):
        buf = chunk % 2
        for j in range(SCH):
            s = chunk * SCH + j
            pltpu.make_async_copy(tab_hbm.at[idx_v.at[s]], rows_v.at[buf, j],
                                  sems[buf]).wait()

    bias = bias_v[...]

    def reduce_chunk(chunk, init):
        buf = chunk % 2

        def reduce_b(b, c):
            p0 = rows_v[buf, 0, b]
            p1 = rows_v[buf, 1, b]
            for j in range(2, SCH, 2):
                p0 = p0 + rows_v[buf, j, b]
                p1 = p1 + rows_v[buf, j + 1, b]
            if init:
                res_v[b] = (p0 + p1) + bias
            else:
                res_v[b] = (p0 + p1) + res_v[b]
            return c

        lax.fori_loop(0, BPW, reduce_b, 0)

    fire(0)
    for chunk in range(NCHUNK):
        if chunk + 1 < NCHUNK:
            fire(chunk + 1)
        drain(chunk)
        reduce_chunk(chunk, init=(chunk == 0))
    pltpu.sync_copy(res_v, out_hbm.at[pl.ds(base, BPW)])


def kernel(x, emb, W1, b1, W2, b2):
    x = x.astype(jnp.int32)
    w2p = jnp.pad(W2, ((0, DPAD - NUM_CLASS), (0, 0)))
    bias = jnp.pad(W2 @ b1 + b2, (0, DPAD - NUM_CLASS))
    tab = _fold_table(emb.T, W1, w2p)
    return _pool_kernel(tab, x, bias)[:, :NUM_CLASS]
